# baseline (speedup 1.0000x reference)
import functools
from functools import partial

import jax
import jax.numpy as jnp
from jax import lax
from jax.experimental import pallas as pl
from jax.experimental.pallas import tpu as pltpu
from jax.experimental.pallas import tpu_sc as plsc

NC = 2
NS = 16
NW = NC * NS
K = 128


def _sc_degree(dst_r, np_rows):
  C = dst_r.shape[1]
  rows_pt = np_rows // NS
  mesh = plsc.VectorSubcoreMesh(core_axis_name="c", subcore_axis_name="s")

  @functools.partial(
      pl.kernel,
      out_type=jax.ShapeDtypeStruct((NC, np_rows), jnp.float32),
      mesh=mesh,
      scratch_types=[
          pltpu.VMEM_SHARED((np_rows,), jnp.float32),
          pltpu.VMEM((C, K), jnp.int32),
          pltpu.VMEM((K,), jnp.float32),
          pltpu.VMEM((rows_pt,), jnp.float32),
      ],
  )
  def deg_kernel(dst_hbm, out_hbm, deg_sh, idx_v, ones_v, zero_v):
    c = lax.axis_index("c")
    s = lax.axis_index("s")
    wid = c * NS + s
    for i in range(K // 16):
      ones_v[pl.ds(i * 16, 16)] = jnp.ones((16,), jnp.float32)
    for i in range(rows_pt // 16):
      zero_v[pl.ds(i * 16, 16)] = jnp.zeros((16,), jnp.float32)
    pltpu.sync_copy(zero_v, deg_sh.at[pl.ds(s * rows_pt, rows_pt)])
    pltpu.sync_copy(dst_hbm.at[wid], idx_v)
    plsc.subcore_barrier()

    def body(j, carry):
      pltpu.sync_copy(ones_v, deg_sh.at[idx_v.at[j]], add=True)
      return carry

    lax.fori_loop(0, C, body, 0, unroll=False)
    plsc.subcore_barrier()
    pltpu.sync_copy(deg_sh.at[pl.ds(s * rows_pt, rows_pt)],
                    out_hbm.at[c, pl.ds(s * rows_pt, rows_pt)])

  return deg_kernel(dst_r)


def _sc_scatter(hp, src_r, dst_r, zeros2d, np_rows, d):
  C = src_r.shape[1]
  rows_pt = np_rows // NS
  mesh = plsc.VectorSubcoreMesh(core_axis_name="c", subcore_axis_name="s")

  @functools.partial(
      pl.kernel,
      out_type=jax.ShapeDtypeStruct((NC, np_rows, d), jnp.float32),
      mesh=mesh,
      scratch_types=[
          pltpu.VMEM_SHARED((np_rows, d), jnp.float32),
          pltpu.VMEM((C, K), jnp.int32),
          pltpu.VMEM((4, K), jnp.int32),
          pltpu.VMEM((2, K, d), jnp.float32),
          pltpu.SemaphoreType.DMA,
          pltpu.SemaphoreType.DMA,
          pltpu.SemaphoreType.DMA,
          pltpu.SemaphoreType.DMA,
          [pltpu.SemaphoreType.DMA] * 4,
      ],
  )
  def scat_kernel(hp_hbm, src_hbm, dst_hbm, zero_hbm, out_hbm,
                  acc_sh, src_v, didx_v, rows_v, g0, g1, s0, s1, dsems):
    c = lax.axis_index("c")
    s = lax.axis_index("s")
    wid = c * NS + s
    row_slice = pl.ds(s * rows_pt, rows_pt)

    @pl.when(c == 0)
    def _():
      pltpu.sync_copy(hp_hbm.at[row_slice], acc_sh.at[row_slice])

    @pl.when(c != 0)
    def _():
      pltpu.sync_copy(zero_hbm.at[row_slice], acc_sh.at[row_slice])

    pltpu.sync_copy(src_hbm.at[wid], src_v)
    plsc.subcore_barrier()

    gsems = (g0, g1)
    ssems = (s0, s1)

    def wait_gather(p):
      pltpu.make_async_copy(
          hp_hbm.at[src_v.at[0]], rows_v.at[p], gsems[p]).wait()

    def wait_scatter(p):
      pltpu.make_async_copy(
          rows_v.at[0], acc_sh.at[didx_v.at[0]], ssems[p]).wait()

    def wait_didx(q):
      pltpu.make_async_copy(
          dst_hbm.at[wid, 0], didx_v.at[q], dsems[q]).wait()

    for q in range(3):
      pltpu.async_copy(dst_hbm.at[wid, q], didx_v.at[q], dsems[q])
    pltpu.async_copy(hp_hbm.at[src_v.at[0]], rows_v.at[0], g0)

    @pl.loop(0, C, step=4)
    def _(j):
      for u in range(4):
        jj = j + u
        p = u % 2
        if u == 0:
          @pl.when(j > 0)
          def _():
            wait_scatter(1)
        else:
          wait_scatter(1 - p)
        nxt = jj + 1
        if u < 3:
          pltpu.async_copy(hp_hbm.at[src_v.at[nxt]], rows_v.at[1 - p],
                           gsems[1 - p])
        else:
          @pl.when(nxt < C)
          def _():
            pltpu.async_copy(hp_hbm.at[src_v.at[nxt]], rows_v.at[1 - p],
                             gsems[1 - p])
        nid = jj + 3

        @pl.when(nid < C)
        def _():
          pltpu.async_copy(dst_hbm.at[wid, nid], didx_v.at[(u + 3) % 4],
                           dsems[(u + 3) % 4])

        wait_gather(p)
        wait_didx(u)
        pltpu.async_copy(rows_v.at[p], acc_sh.at[didx_v.at[u]], ssems[p],
                         add=True)

    wait_scatter((C - 1) % 2)
    plsc.subcore_barrier()
    pltpu.sync_copy(acc_sh.at[row_slice], out_hbm.at[c, row_slice])

  return scat_kernel(hp, src_r, dst_r, zeros2d)


def _block_dinv(deg_ref, n):
  deg = deg_ref[0] + deg_ref[1] + 1.0
  rid = (pl.program_id(0) * 128
         + lax.broadcasted_iota(jnp.int32, (128, 1), 0))
  return jnp.where(rid < n, lax.rsqrt(deg), 0.0)


def _deg_spec():
  return pl.BlockSpec((NC, 128, 1), lambda i: (0, i, 0))


def _tc_mm_scale(x, w, deg3, n, np_rows, d):
  def body(x_ref, w_ref, deg_ref, out_ref):
    out_ref[...] = jnp.dot(
        x_ref[...], w_ref[...], preferred_element_type=jnp.float32
    ) * _block_dinv(deg_ref, n)

  return pl.pallas_call(
      body,
      grid=(np_rows // 128,),
      in_specs=[
          pl.BlockSpec((128, d), lambda i: (i, 0)),
          pl.BlockSpec((d, d), lambda i: (0, 0)),
          _deg_spec(),
      ],
      out_specs=pl.BlockSpec((128, d), lambda i: (i, 0)),
      out_shape=jax.ShapeDtypeStruct((np_rows, d), jnp.float32),
  )(x, w, deg3)


def _tc_fused_next(acc, deg3, b, w_next, n, np_rows, d):
  def body(acc_ref, deg_ref, b_ref, w_ref, out_ref):
    dinv = _block_dinv(deg_ref, n)
    v = dinv * (acc_ref[0] + acc_ref[1]) + b_ref[...]
    v = jnp.maximum(v, 0.0)
    out_ref[...] = jnp.dot(
        v, w_ref[...], preferred_element_type=jnp.float32
    ) * dinv

  return pl.pallas_call(
      body,
      grid=(np_rows // 128,),
      in_specs=[
          pl.BlockSpec((NC, 128, d), lambda i: (0, i, 0)),
          _deg_spec(),
          pl.BlockSpec((1, d), lambda i: (0, 0)),
          pl.BlockSpec((d, d), lambda i: (0, 0)),
      ],
      out_specs=pl.BlockSpec((128, d), lambda i: (i, 0)),
      out_shape=jax.ShapeDtypeStruct((np_rows, d), jnp.float32),
  )(acc, deg3, b.reshape(1, d), w_next)


def _tc_final(acc, deg3, b, n, d):
  def body(acc_ref, deg_ref, b_ref, out_ref):
    v = _block_dinv(deg_ref, n) * (acc_ref[0] + acc_ref[1]) + b_ref[...]
    ss = jnp.sum(v * v, axis=-1, keepdims=True)
    out_ref[...] = v * lax.rsqrt(jnp.maximum(ss, 1e-24))

  grid = (n + 127) // 128
  return pl.pallas_call(
      body,
      grid=(grid,),
      in_specs=[
          pl.BlockSpec((NC, 128, d), lambda i: (0, i, 0)),
          _deg_spec(),
          pl.BlockSpec((1, d), lambda i: (0, 0)),
      ],
      out_specs=pl.BlockSpec((128, d), lambda i: (i, 0)),
      out_shape=jax.ShapeDtypeStruct((n, d), jnp.float32),
  )(acc, deg3, b.reshape(1, d))


@jax.jit
def kernel(x, edge_index, W1, b1, W2, b2, W3, b3):
  n, d = x.shape
  e = edge_index.shape[1]
  dummy = n
  np_rows = ((n + 1 + NS * 128 - 1) // (NS * 128)) * (NS * 128)

  src = edge_index[0].astype(jnp.int32)
  dst = edge_index[1].astype(jnp.int32)
  C = (e + NW * K - 1) // (NW * K)
  C = ((C + 3) // 4) * 4
  e2 = NW * C * K
  pad = e2 - e
  fill = dummy + (jnp.arange(pad, dtype=jnp.int32) % (np_rows - n))
  src_r = jnp.concatenate([src, fill]).reshape(NW, C, K)
  dst_r = jnp.concatenate([dst, fill]).reshape(NW, C, K)

  x_pad = jnp.zeros((np_rows, d), jnp.float32).at[:n].set(x)
  zeros2d = jnp.zeros((np_rows, d), jnp.float32)

  deg3 = _sc_degree(dst_r, np_rows).reshape(NC, np_rows, 1)

  hp1 = _tc_mm_scale(x_pad, W1, deg3, n, np_rows, d)
  acc1 = _sc_scatter(hp1, src_r, dst_r, zeros2d, np_rows, d)
  hp2 = _tc_fused_next(acc1, deg3, b1, W2, n, np_rows, d)
  acc2 = _sc_scatter(hp2, src_r, dst_r, zeros2d, np_rows, d)
  hp3 = _tc_fused_next(acc2, deg3, b2, W3, n, np_rows, d)
  acc3 = _sc_scatter(hp3, src_r, dst_r, zeros2d, np_rows, d)
  return _tc_final(acc3, deg3, b3, n, d)


# speedup vs baseline: 1.0160x; 1.0160x over previous
import functools
from functools import partial

import jax
import jax.numpy as jnp
from jax import lax
from jax.experimental import pallas as pl
from jax.experimental.pallas import tpu as pltpu
from jax.experimental.pallas import tpu_sc as plsc

NC = 2
NS = 16
NW = NC * NS
K = 128


def _sc_degree(dst_r, np_rows):
  C = dst_r.shape[1]
  rows_pt = np_rows // NS
  mesh = plsc.VectorSubcoreMesh(core_axis_name="c", subcore_axis_name="s")

  @functools.partial(
      pl.kernel,
      out_type=jax.ShapeDtypeStruct((NC, np_rows), jnp.float32),
      mesh=mesh,
      scratch_types=[
          pltpu.VMEM_SHARED((np_rows,), jnp.float32),
          pltpu.VMEM((C, K), jnp.int32),
          pltpu.VMEM((K,), jnp.float32),
          pltpu.VMEM((rows_pt,), jnp.float32),
      ],
  )
  def deg_kernel(dst_hbm, out_hbm, deg_sh, idx_v, ones_v, zero_v):
    c = lax.axis_index("c")
    s = lax.axis_index("s")
    wid = c * NS + s
    for i in range(K // 16):
      ones_v[pl.ds(i * 16, 16)] = jnp.ones((16,), jnp.float32)
    for i in range(rows_pt // 16):
      zero_v[pl.ds(i * 16, 16)] = jnp.zeros((16,), jnp.float32)
    pltpu.sync_copy(zero_v, deg_sh.at[pl.ds(s * rows_pt, rows_pt)])
    pltpu.sync_copy(dst_hbm.at[wid], idx_v)
    plsc.subcore_barrier()

    def body(j, carry):
      pltpu.sync_copy(ones_v, deg_sh.at[idx_v.at[j]], add=True)
      return carry

    lax.fori_loop(0, C, body, 0, unroll=False)
    plsc.subcore_barrier()
    pltpu.sync_copy(deg_sh.at[pl.ds(s * rows_pt, rows_pt)],
                    out_hbm.at[c, pl.ds(s * rows_pt, rows_pt)])

  return deg_kernel(dst_r)


def _sc_scatter(hp, src_r, dst_r, zeros2d, np_rows, d):
  C = src_r.shape[1]
  rows_pt = np_rows // NS
  mesh = plsc.VectorSubcoreMesh(core_axis_name="c", subcore_axis_name="s")

  @functools.partial(
      pl.kernel,
      out_type=jax.ShapeDtypeStruct((NC, np_rows, d), jnp.float32),
      mesh=mesh,
      scratch_types=[
          pltpu.VMEM_SHARED((np_rows, d), jnp.float32),
          pltpu.VMEM((C, K), jnp.int32),
          pltpu.VMEM((4, K), jnp.int32),
          pltpu.VMEM((2, K, d), jnp.float32),
          pltpu.SemaphoreType.DMA,
          pltpu.SemaphoreType.DMA,
          pltpu.SemaphoreType.DMA,
          pltpu.SemaphoreType.DMA,
          pltpu.SemaphoreType.DMA,
          [pltpu.SemaphoreType.DMA] * 4,
      ],
  )
  def scat_kernel(hp_hbm, src_hbm, dst_hbm, zero_hbm, out_hbm,
                  acc_sh, src_v, didx_v, rows_v, g0, g1, s0, s1, isem, dsems):
    c = lax.axis_index("c")
    s = lax.axis_index("s")
    wid = c * NS + s
    row_slice = pl.ds(s * rows_pt, rows_pt)

    @pl.when(c == 0)
    def _():
      pltpu.async_copy(hp_hbm.at[row_slice], acc_sh.at[row_slice], isem)

    @pl.when(c != 0)
    def _():
      pltpu.async_copy(zero_hbm.at[row_slice], acc_sh.at[row_slice], isem)

    pltpu.sync_copy(src_hbm.at[wid], src_v)

    gsems = (g0, g1)
    ssems = (s0, s1)

    def wait_gather(p):
      pltpu.make_async_copy(
          hp_hbm.at[src_v.at[0]], rows_v.at[p], gsems[p]).wait()

    def wait_scatter(p):
      pltpu.make_async_copy(
          rows_v.at[0], acc_sh.at[didx_v.at[0]], ssems[p]).wait()

    def wait_didx(q):
      pltpu.make_async_copy(
          dst_hbm.at[wid, 0], didx_v.at[q], dsems[q]).wait()

    for q in range(3):
      pltpu.async_copy(dst_hbm.at[wid, q], didx_v.at[q], dsems[q])
    pltpu.async_copy(hp_hbm.at[src_v.at[0]], rows_v.at[0], g0)
    pltpu.make_async_copy(
        hp_hbm.at[row_slice], acc_sh.at[row_slice], isem).wait()
    plsc.subcore_barrier()

    @pl.loop(0, C, step=4)
    def _(j):
      for u in range(4):
        jj = j + u
        p = u % 2
        if u == 0:
          @pl.when(j > 0)
          def _():
            wait_scatter(1)
        else:
          wait_scatter(1 - p)
        nxt = jj + 1
        if u < 3:
          pltpu.async_copy(hp_hbm.at[src_v.at[nxt]], rows_v.at[1 - p],
                           gsems[1 - p])
        else:
          @pl.when(nxt < C)
          def _():
            pltpu.async_copy(hp_hbm.at[src_v.at[nxt]], rows_v.at[1 - p],
                             gsems[1 - p])
        nid = jj + 3

        @pl.when(nid < C)
        def _():
          pltpu.async_copy(dst_hbm.at[wid, nid], didx_v.at[(u + 3) % 4],
                           dsems[(u + 3) % 4])

        wait_gather(p)
        wait_didx(u)
        pltpu.async_copy(rows_v.at[p], acc_sh.at[didx_v.at[u]], ssems[p],
                         add=True)

    wait_scatter((C - 1) % 2)
    plsc.subcore_barrier()
    pltpu.sync_copy(acc_sh.at[row_slice], out_hbm.at[c, row_slice])

  return scat_kernel(hp, src_r, dst_r, zeros2d)


def _block_dinv(deg_ref, n):
  deg = deg_ref[0] + deg_ref[1] + 1.0
  rid = (pl.program_id(0) * 128
         + lax.broadcasted_iota(jnp.int32, (128, 1), 0))
  return jnp.where(rid < n, lax.rsqrt(deg), 0.0)


def _deg_spec():
  return pl.BlockSpec((NC, 128, 1), lambda i: (0, i, 0))


def _tc_mm_scale(x, w, deg3, n, np_rows, d):
  def body(x_ref, w_ref, deg_ref, out_ref):
    out_ref[...] = jnp.dot(
        x_ref[...], w_ref[...], preferred_element_type=jnp.float32
    ) * _block_dinv(deg_ref, n)

  return pl.pallas_call(
      body,
      grid=(np_rows // 128,),
      in_specs=[
          pl.BlockSpec((128, d), lambda i: (i, 0)),
          pl.BlockSpec((d, d), lambda i: (0, 0)),
          _deg_spec(),
      ],
      out_specs=pl.BlockSpec((128, d), lambda i: (i, 0)),
      out_shape=jax.ShapeDtypeStruct((np_rows, d), jnp.float32),
  )(x, w, deg3)


def _tc_fused_next(acc, deg3, b, w_next, n, np_rows, d):
  def body(acc_ref, deg_ref, b_ref, w_ref, out_ref):
    dinv = _block_dinv(deg_ref, n)
    v = dinv * (acc_ref[0] + acc_ref[1]) + b_ref[...]
    v = jnp.maximum(v, 0.0)
    out_ref[...] = jnp.dot(
        v, w_ref[...], preferred_element_type=jnp.float32
    ) * dinv

  return pl.pallas_call(
      body,
      grid=(np_rows // 128,),
      in_specs=[
          pl.BlockSpec((NC, 128, d), lambda i: (0, i, 0)),
          _deg_spec(),
          pl.BlockSpec((1, d), lambda i: (0, 0)),
          pl.BlockSpec((d, d), lambda i: (0, 0)),
      ],
      out_specs=pl.BlockSpec((128, d), lambda i: (i, 0)),
      out_shape=jax.ShapeDtypeStruct((np_rows, d), jnp.float32),
  )(acc, deg3, b.reshape(1, d), w_next)


def _tc_final(acc, deg3, b, n, d):
  def body(acc_ref, deg_ref, b_ref, out_ref):
    v = _block_dinv(deg_ref, n) * (acc_ref[0] + acc_ref[1]) + b_ref[...]
    ss = jnp.sum(v * v, axis=-1, keepdims=True)
    out_ref[...] = v * lax.rsqrt(jnp.maximum(ss, 1e-24))

  grid = (n + 127) // 128
  return pl.pallas_call(
      body,
      grid=(grid,),
      in_specs=[
          pl.BlockSpec((NC, 128, d), lambda i: (0, i, 0)),
          _deg_spec(),
          pl.BlockSpec((1, d), lambda i: (0, 0)),
      ],
      out_specs=pl.BlockSpec((128, d), lambda i: (i, 0)),
      out_shape=jax.ShapeDtypeStruct((n, d), jnp.float32),
  )(acc, deg3, b.reshape(1, d))


@jax.jit
def kernel(x, edge_index, W1, b1, W2, b2, W3, b3):
  n, d = x.shape
  e = edge_index.shape[1]
  dummy = n
  np_rows = ((n + 1 + NS * 128 - 1) // (NS * 128)) * (NS * 128)

  src = edge_index[0].astype(jnp.int32)
  dst = edge_index[1].astype(jnp.int32)
  C = (e + NW * K - 1) // (NW * K)
  C = ((C + 3) // 4) * 4
  e2 = NW * C * K
  pad = e2 - e
  fill = dummy + (jnp.arange(pad, dtype=jnp.int32) % (np_rows - n))
  src_r = jnp.concatenate([src, fill]).reshape(NW, C, K)
  dst_r = jnp.concatenate([dst, fill]).reshape(NW, C, K)

  x_pad = jnp.zeros((np_rows, d), jnp.float32).at[:n].set(x)
  zeros2d = jnp.zeros((np_rows, d), jnp.float32)

  deg3 = _sc_degree(dst_r, np_rows).reshape(NC, np_rows, 1)

  hp1 = _tc_mm_scale(x_pad, W1, deg3, n, np_rows, d)
  acc1 = _sc_scatter(hp1, src_r, dst_r, zeros2d, np_rows, d)
  hp2 = _tc_fused_next(acc1, deg3, b1, W2, n, np_rows, d)
  acc2 = _sc_scatter(hp2, src_r, dst_r, zeros2d, np_rows, d)
  hp3 = _tc_fused_next(acc2, deg3, b2, W3, n, np_rows, d)
  acc3 = _sc_scatter(hp3, src_r, dst_r, zeros2d, np_rows, d)
  return _tc_final(acc3, deg3, b3, n, d)
